# baseline (speedup 1.0000x reference)
import functools

import jax
import jax.numpy as jnp
from jax import lax
from jax.experimental import pallas as pl
from jax.experimental.pallas import tpu as pltpu
from jax.experimental.pallas import tpu_sc as plsc

N = 10000
E = 320000
D = 128
A = 32
NPAD = 10240
WAUG = 144
NC = 2
NS = 16
NW = NC * NS
EPW = E // NW
C = 80
NCHUNK = EPW // C
SENT = -3.0e38


def _tc1_body(inp, we, be, wl, wr, wres, xl, xr, xres):
    x = jnp.maximum(jnp.dot(inp[...], we[...],
                            preferred_element_type=jnp.float32) + be[0:1], 0.0)
    xl[...] = jnp.dot(x, wl[...], preferred_element_type=jnp.float32)
    xr[...] = jnp.dot(x, wr[...], preferred_element_type=jnp.float32)
    xres[...] = jnp.dot(x, wres[...], preferred_element_type=jnp.float32)


def _tc2_body(eaT, wedge, out):
    out[...] = lax.dot_general(eaT[...], wedge[...],
                               (((0,), (0,)), ((), ())),
                               preferred_element_type=jnp.float32)


def _tc3_body(parts, denr, xresr, bg, wq, bq, qo, ho):
    num = parts[0] + parts[1]
    den = lax.dot_general(denr[...], jnp.ones((NW, 1), jnp.float32),
                          (((0,), (0,)), ((), ())),
                          preferred_element_type=jnp.float32)
    out = num / (den + 1e-16)
    h = out + xresr[...] + bg[0:1]
    ho[...] = h
    qo[...] = jnp.dot(h, wq[...], preferred_element_type=jnp.float32) + bq[0:1]


def _lane():
    return lax.iota(jnp.int32, 16)


def _take16(x, idx):
    dnums = lax.GatherDimensionNumbers(
        offset_dims=(), collapsed_slice_dims=(0,), start_index_map=(0,))
    return lax.gather(x, idx[:, None], dnums, slice_sizes=(1,),
                      mode=lax.GatherScatterMode.PROMISE_IN_BOUNDS)


def _shift_up(x, k):
    return _take16(x, jnp.maximum(_lane() - k, 0))


def _seg_scan(d, x, op):
    for k in (1, 2, 4, 8):
        xs = _shift_up(x, k)
        ds = _shift_up(d, k)
        x = jnp.where((ds == d) & (_lane() >= k), op(x, xs), x)
    return x


def _run_last(d):
    nxt = _take16(d, jnp.minimum(_lane() + 1, 15))
    return (d != nxt) | (_lane() == 15)


def _sca_body(xl_hbm, xr_hbm, eemb_hbm, src_hbm, dst_hbm, att_hbm,
              sdl_hbm, smaxc_hbm,
              srcf, dstf, rl_v, rr_v, em_v, sdlrow_v, att_v, smax_v,
              comb_v, combo_v, shared_max,
              sl0, sr0, se0, sl1, sr1, se1, sw0, sw1):
    cid = lax.axis_index("c")
    sid = lax.axis_index("s")
    wid = cid * NS + sid

    pltpu.sync_copy(att_hbm, att_v)
    att_vecs = [att_v[pl.ds(j * 16, 16)] for j in range(D // 16)]
    pltpu.sync_copy(src_hbm.at[wid], srcf)
    pltpu.sync_copy(dst_hbm.at[wid], dstf)

    def init_body(i, _):
        smax_v[pl.ds(i * 16, 16)] = jnp.full((16,), SENT, jnp.float32)
        return 0
    lax.fori_loop(0, NPAD // 16, init_body, 0)

    sems = ((sl0, sr0, se0), (sl1, sr1, se1))
    bufs = ((rl_v.at[0], rr_v.at[0], em_v.at[0]),
            (rl_v.at[1], rr_v.at[1], em_v.at[1]))

    def issue(k, b):
        base = k * C
        sms, bfs = sems[b], bufs[b]
        pltpu.async_copy(xl_hbm.at[srcf.at[pl.ds(base, C)]], bfs[0], sms[0])
        pltpu.async_copy(xr_hbm.at[dstf.at[pl.ds(base, C)]], bfs[1], sms[1])
        pltpu.async_copy(eemb_hbm.at[pl.ds(wid * EPW + base, C)],
                         bfs[2], sms[2])

    def wait(b):
        sms, bfs = sems[b], bufs[b]
        for t in range(3):
            pltpu.make_async_copy(eemb_hbm.at[pl.ds(0, C)],
                                  bfs[t], sms[t]).wait()

    lane = lax.iota(jnp.int32, 16)
    wsems = (sw0, sw1)

    def compute(k, b, wait_w):
        rl, rr, em = bufs[b]
        base = k * C
        if wait_w is not None:
            @pl.when(wait_w)
            def _():
                pltpu.make_async_copy(sdl_hbm.at[0], sdlrow_v.at[b],
                                      wsems[b]).wait()

        def block_body(v, _):
            lvec = jnp.zeros((16,), jnp.float32)
            for l in range(16):
                c = v * 16 + l
                acc = jnp.zeros((16,), jnp.float32)
                for j in range(D // 16):
                    sl = pl.ds(j * 16, 16)
                    m = rl[c, sl] + rr[c, sl] + em[c, sl]
                    a = jnp.where(m >= 0.0, m, 0.2 * m)
                    acc = acc + a * att_vecs[j]
                lvec = jnp.where(lane == l, jnp.sum(acc), lvec)
            sl16 = pl.ds(v * 16, 16)
            sdlrow_v[b, sl16] = srcf[pl.ds(base + v * 16, 16)]
            dvec = dstf[pl.ds(base + v * 16, 16)]
            sdlrow_v[b, pl.ds(C + v * 16, 16)] = dvec
            sdlrow_v[b, pl.ds(2 * C + v * 16, 16)] = plsc.bitcast(
                lvec, jnp.int32)
            s_d, s_l = plsc.sort_key_val(dvec, lvec)
            segm = _seg_scan(s_d, s_l, jnp.maximum)
            last = _run_last(s_d)
            cur = plsc.load_gather(smax_v, [s_d])
            plsc.store_scatter(smax_v, [s_d], jnp.maximum(cur, segm),
                               mask=last)
            return 0
        lax.fori_loop(0, C // 16, block_body, 0)
        pltpu.async_copy(sdlrow_v.at[b], sdl_hbm.at[wid * NCHUNK + k],
                         wsems[b])

    issue(0, 0)

    def pipe_body(i, _):
        k0 = 2 * i
        issue(k0 + 1, 1)
        wait(0)
        compute(k0, 0, k0 >= 2)
        issue(k0 + 2, 0)
        wait(1)
        compute(k0 + 1, 1, k0 >= 1)
        return 0
    lax.fori_loop(0, (NCHUNK - 1) // 2, pipe_body, 0)
    wait(0)
    compute(NCHUNK - 1, 0, jnp.bool_(True))
    pltpu.make_async_copy(sdl_hbm.at[0], sdlrow_v.at[0], sw0).wait()
    pltpu.make_async_copy(sdl_hbm.at[0], sdlrow_v.at[1], sw1).wait()

    pltpu.sync_copy(smax_v, shared_max.at[sid])
    plsc.subcore_barrier()
    per = NPAD // NS
    for j in range(NS):
        pltpu.sync_copy(shared_max.at[j, pl.ds(sid * per, per)], comb_v.at[j])

    def comb_body(v, _):
        sl = pl.ds(v * 16, 16)
        m = comb_v[0, sl]
        for j in range(1, NS):
            m = jnp.maximum(m, comb_v[j, sl])
        combo_v[sl] = m
        return 0
    lax.fori_loop(0, per // 16, comb_body, 0)
    pltpu.sync_copy(combo_v, smaxc_hbm.at[cid, pl.ds(sid * per, per)])


def _scb_body(xl_hbm, sdl_hbm, smaxc_hbm, smaxr_hbm,
              parts_hbm, denp_hbm,
              sdl_v, src_v, dstA, dstB, ex_v, rl_v, segA, den_v, acc_sh,
              si0, si1, s0, s1, sc0, sc1):
    cid = lax.axis_index("c")
    sid = lax.axis_index("s")
    wid = cid * NS + sid

    pltpu.sync_copy(smaxc_hbm.at[0], segA)
    pltpu.sync_copy(smaxr_hbm.at[1], rl_v.at[1])

    def comb_body(r, _):
        for j in range(D // 16):
            sl16 = pl.ds(r * D + j * 16, 16)
            segA[sl16] = jnp.maximum(segA[sl16],
                                     rl_v[1, r, pl.ds(j * 16, 16)])
        return 0
    lax.fori_loop(0, C, comb_body, 0)

    def zden_body(v, _):
        den_v[pl.ds(v * 16, 16)] = jnp.zeros((16,), jnp.float32)
        return 0
    lax.fori_loop(0, NPAD // 16, zden_body, 0)

    lane16 = lax.iota(jnp.int32, 16)

    def zrow_body(v, _):
        for j in range(D // 16):
            rl_v[0, v, pl.ds(j * 16, 16)] = jnp.zeros((16,), jnp.float32)
        return 0
    lax.fori_loop(0, C, zrow_body, 0)
    for t in range(NPAD // NS // C):
        for v in range(C // 16):
            dstA[pl.ds(v * 16, 16)] = (sid * (NPAD // NS) + t * C
                                       + v * 16 + lane16)
        pltpu.sync_copy(rl_v.at[0], acc_sh.at[dstA])
    plsc.subcore_barrier()

    isems = (si0, si1)
    rsems = (s0, s1)
    csems = (sc0, sc1)
    dst_bufs = (dstA, dstB)

    def issue_sdl(k, b):
        pltpu.async_copy(sdl_hbm.at[wid * NCHUNK + k], sdl_v.at[b], isems[b])

    def wait_sdl(b):
        pltpu.make_async_copy(sdl_hbm.at[0], sdl_v.at[b],
                              isems[b]).wait()

    def issue_rows(k, b, bi):
        for v in range(C // 16):
            src_v[pl.ds(v * 16, 16)] = sdl_v[bi, pl.ds(v * 16, 16)]
        pltpu.async_copy(xl_hbm.at[src_v], rl_v.at[b], rsems[b])

    def wait_rows(b):
        pltpu.make_async_copy(xl_hbm.at[pl.ds(0, C)], rl_v.at[b],
                              rsems[b]).wait()

    def step(k, p, do_sdl, do_next, wait_sc):
        dst_p = dst_bufs[p]
        if wait_sc is not None:
            @pl.when(wait_sc)
            def _():
                pltpu.make_async_copy(xl_hbm.at[pl.ds(0, C)],
                                      rl_v.at[p], csems[p]).wait()

        def ex_body(v, _):
            sl = pl.ds(v * 16, 16)
            dvec = sdl_v[p, pl.ds(C + v * 16, 16)]
            dst_p[sl] = dvec
            mx = plsc.load_gather(segA, [dvec])
            lg = plsc.bitcast(sdl_v[p, pl.ds(2 * C + v * 16, 16)],
                              jnp.float32)
            ex = jnp.exp(lg - mx)
            ex_v[sl] = ex
            s_d, s_e = plsc.sort_key_val(dvec, ex)
            sege = _seg_scan(s_d, s_e, lax.add)
            plsc.addupdate_scatter(den_v, [s_d], sege, mask=_run_last(s_d))
            return 0
        lax.fori_loop(0, C // 16, ex_body, 0)
        wait_rows(p)
        if do_sdl is not None:
            @pl.when(do_sdl)
            def _():
                issue_sdl(k + 2, p)
        if do_next:
            wait_sdl(1 - p)
            issue_rows(k + 1, 1 - p, 1 - p)

        def edge_block(v, _):
            exvec = ex_v[pl.ds(v * 16, 16)]
            for l in range(16):
                c = v * 16 + l
                e = exvec[l]
                for j in range(D // 16):
                    sl = pl.ds(j * 16, 16)
                    rl_v[p, c, sl] = rl_v[p, c, sl] * e
            return 0
        lax.fori_loop(0, C // 16, edge_block, 0)

        pltpu.async_copy(rl_v.at[p], acc_sh.at[dst_p], csems[p], add=True)

    issue_sdl(0, 0)
    wait_sdl(0)
    issue_rows(0, 0, 0)
    issue_sdl(1, 1)

    def pipe_body(i, _):
        k0 = 2 * i
        step(k0, 0, jnp.bool_(True), True, k0 >= 2)
        step(k0 + 1, 1, (k0 + 3) < NCHUNK, True, k0 >= 1)
        return 0
    lax.fori_loop(0, (NCHUNK - 1) // 2, pipe_body, 0)
    step(NCHUNK - 1, 0, None, False, jnp.bool_(True))

    pltpu.make_async_copy(xl_hbm.at[pl.ds(0, C)], rl_v.at[0], sc0).wait()
    pltpu.make_async_copy(xl_hbm.at[pl.ds(0, C)], rl_v.at[1], sc1).wait()

    pltpu.sync_copy(den_v, denp_hbm.at[wid])
    plsc.subcore_barrier()

    @pl.when(sid == 0)
    def _():
        pltpu.sync_copy(acc_sh, parts_hbm.at[cid])


@jax.jit
def kernel(inputs, hidden_states, edge_index, edge_attr, W_e, b_e, W_ih, b_ih,
           W_hh, b_hh, W_l, W_r, W_edge, att, b_gat, W_res, W_q, b_q):
    del hidden_states, W_ih, b_ih, W_hh, b_hh
    f32 = jnp.float32
    src = edge_index[0]
    dst = edge_index[1]

    RB = 2000
    xl, xr, xres = pl.pallas_call(
        _tc1_body,
        grid=(N // RB,),
        in_specs=[
            pl.BlockSpec((RB, D), lambda i: (i, 0)),
            pl.BlockSpec((D, D), lambda i: (0, 0)),
            pl.BlockSpec((8, D), lambda i: (0, 0)),
            pl.BlockSpec((D, D), lambda i: (0, 0)),
            pl.BlockSpec((D, D), lambda i: (0, 0)),
            pl.BlockSpec((D, D), lambda i: (0, 0)),
        ],
        out_specs=[
            pl.BlockSpec((RB, D), lambda i: (i, 0)),
            pl.BlockSpec((RB, D), lambda i: (i, 0)),
            pl.BlockSpec((RB, D), lambda i: (i, 0)),
        ],
        out_shape=[
            jax.ShapeDtypeStruct((N, D), f32),
            jax.ShapeDtypeStruct((N, D), f32),
            jax.ShapeDtypeStruct((N, D), f32),
        ],
    )(inputs, W_e, jnp.broadcast_to(b_e.reshape(1, D), (8, D)),
      W_l, W_r, W_res)

    EB = 16000
    e_emb = pl.pallas_call(
        _tc2_body,
        grid=(E // EB,),
        in_specs=[
            pl.BlockSpec((5, EB), lambda i: (0, i)),
            pl.BlockSpec((5, D), lambda i: (0, 0)),
        ],
        out_specs=pl.BlockSpec((EB, D), lambda i: (i, 0)),
        out_shape=jax.ShapeDtypeStruct((E, D), f32),
    )(jnp.transpose(edge_attr), W_edge)

    mesh = plsc.VectorSubcoreMesh(core_axis_name="c", subcore_axis_name="s")
    sc_params = pltpu.CompilerParams(needs_layout_passes=False)

    src2 = src.reshape(NW, EPW)
    dst2 = dst.reshape(NW, EPW)
    sdl, smaxc = pl.kernel(
        _sca_body,
        compiler_params=sc_params,
        out_type=(jax.ShapeDtypeStruct((NW * NCHUNK, 3 * C), jnp.int32),
                  jax.ShapeDtypeStruct((NC, NPAD), f32)),
        mesh=mesh,
        scratch_types=[
            pltpu.VMEM((EPW,), jnp.int32),
            pltpu.VMEM((EPW,), jnp.int32),
            pltpu.VMEM((2, C, D), f32),
            pltpu.VMEM((2, C, D), f32),
            pltpu.VMEM((2, C, D), f32),
            pltpu.VMEM((2, 3 * C), jnp.int32),
            pltpu.VMEM((D,), f32),
            pltpu.VMEM((NPAD,), f32),
            pltpu.VMEM((NS, NPAD // NS), f32),
            pltpu.VMEM((NPAD // NS,), f32),
            pltpu.VMEM_SHARED((NS, NPAD), f32),
            pltpu.SemaphoreType.DMA,
            pltpu.SemaphoreType.DMA,
            pltpu.SemaphoreType.DMA,
            pltpu.SemaphoreType.DMA,
            pltpu.SemaphoreType.DMA,
            pltpu.SemaphoreType.DMA,
            pltpu.SemaphoreType.DMA,
            pltpu.SemaphoreType.DMA,
        ],
    )(xl, xr, e_emb, src2, dst2, att)

    parts, denp = pl.kernel(
        _scb_body,
        compiler_params=sc_params,
        out_type=(jax.ShapeDtypeStruct((NC, NPAD, D), f32),
                  jax.ShapeDtypeStruct((NW, NPAD), f32)),
        mesh=mesh,
        scratch_types=[
            pltpu.VMEM((2, 3 * C), jnp.int32),
            pltpu.VMEM((C,), jnp.int32),
            pltpu.VMEM((C,), jnp.int32),
            pltpu.VMEM((C,), jnp.int32),
            pltpu.VMEM((C,), f32),
            pltpu.VMEM((2, C, D), f32),
            pltpu.VMEM((NPAD,), f32),
            pltpu.VMEM((NPAD,), f32),
            pltpu.VMEM_SHARED((NPAD, D), f32),
            pltpu.SemaphoreType.DMA,
            pltpu.SemaphoreType.DMA,
            pltpu.SemaphoreType.DMA,
            pltpu.SemaphoreType.DMA,
            pltpu.SemaphoreType.DMA,
            pltpu.SemaphoreType.DMA,
        ],
    )(xl, sdl, smaxc, smaxc.reshape(NC, C, D))

    RB3 = 1280
    qp, hp = pl.pallas_call(
        _tc3_body,
        grid=(NPAD // RB3,),
        in_specs=[
            pl.BlockSpec((NC, RB3, D), lambda i: (0, i, 0)),
            pl.BlockSpec((NW, RB3), lambda i: (0, i)),
            pl.BlockSpec((RB3, D), lambda i: (i, 0)),
            pl.BlockSpec((8, D), lambda i: (0, 0)),
            pl.BlockSpec((D, A), lambda i: (0, 0)),
            pl.BlockSpec((8, A), lambda i: (0, 0)),
        ],
        out_specs=[
            pl.BlockSpec((RB3, A), lambda i: (i, 0)),
            pl.BlockSpec((RB3, D), lambda i: (i, 0)),
        ],
        out_shape=[
            jax.ShapeDtypeStruct((NPAD, A), f32),
            jax.ShapeDtypeStruct((NPAD, D), f32),
        ],
    )(parts, denp, jnp.pad(xres, ((0, NPAD - N), (0, 0))),
      jnp.broadcast_to(b_gat.reshape(1, D), (8, D)),
      W_q, jnp.broadcast_to(b_q.reshape(1, A), (8, A)))

    return (qp[:N], hp[:N])


# speedup vs baseline: 1.0165x; 1.0165x over previous
import functools

import jax
import jax.numpy as jnp
from jax import lax
from jax.experimental import pallas as pl
from jax.experimental.pallas import tpu as pltpu
from jax.experimental.pallas import tpu_sc as plsc

N = 10000
E = 320000
D = 128
A = 32
NPAD = 10240
WAUG = 144
NC = 2
NS = 16
NW = NC * NS
EPW = E // NW
C = 80
NCHUNK = EPW // C
SENT = -3.0e38


def _tc1_body(inp, we, be, wl, wr, wres, xl, xr, xres):
    x = jnp.maximum(jnp.dot(inp[...], we[...],
                            preferred_element_type=jnp.float32) + be[0:1], 0.0)
    xl[...] = jnp.dot(x, wl[...], preferred_element_type=jnp.float32)
    xr[...] = jnp.dot(x, wr[...], preferred_element_type=jnp.float32)
    xres[...] = jnp.dot(x, wres[...], preferred_element_type=jnp.float32)


def _tc2_body(eaT, wedge, out):
    out[...] = lax.dot_general(eaT[...], wedge[...],
                               (((0,), (0,)), ((), ())),
                               preferred_element_type=jnp.float32)


def _tc3_body(parts, denr, xresr, bg, wq, bq, qo, ho):
    num = parts[0] + parts[1]
    den = lax.dot_general(denr[...], jnp.ones((NW, 1), jnp.float32),
                          (((0,), (0,)), ((), ())),
                          preferred_element_type=jnp.float32)
    out = num / (den + 1e-16)
    h = out + xresr[...] + bg[0:1]
    ho[...] = h
    qo[...] = jnp.dot(h, wq[...], preferred_element_type=jnp.float32) + bq[0:1]


def _lane():
    return lax.iota(jnp.int32, 16)


def _take16(x, idx):
    dnums = lax.GatherDimensionNumbers(
        offset_dims=(), collapsed_slice_dims=(0,), start_index_map=(0,))
    return lax.gather(x, idx[:, None], dnums, slice_sizes=(1,),
                      mode=lax.GatherScatterMode.PROMISE_IN_BOUNDS)


def _shift_up(x, k):
    return _take16(x, jnp.maximum(_lane() - k, 0))


def _seg_scan(d, x, op):
    for k in (1, 2, 4, 8):
        xs = _shift_up(x, k)
        ds = _shift_up(d, k)
        x = jnp.where((ds == d) & (_lane() >= k), op(x, xs), x)
    return x


def _run_last(d):
    nxt = _take16(d, jnp.minimum(_lane() + 1, 15))
    return (d != nxt) | (_lane() == 15)


def _sca_body(xl_hbm, xr_hbm, eemb3_hbm, src_hbm, dst_hbm, att_hbm,
              sdl_hbm, smaxc_hbm,
              srcf, dstf, rre_v, sdlrow_v, att_v, smax_v,
              comb_v, combo_v, shared_max,
              sl0, sl1, sw0, sw1):
    cid = lax.axis_index("c")
    sid = lax.axis_index("s")
    wid = cid * NS + sid

    pltpu.sync_copy(att_hbm, att_v)
    att_vecs = [att_v[pl.ds(j * 16, 16)] for j in range(D // 16)]
    pltpu.sync_copy(src_hbm.at[wid], srcf)
    pltpu.sync_copy(dst_hbm.at[wid], dstf)

    def init_body(i, _):
        smax_v[pl.ds(i * 16, 16)] = jnp.full((16,), SENT, jnp.float32)
        return 0
    lax.fori_loop(0, NPAD // 16, init_body, 0)

    sems = (sl0, sl1)
    bufs = ((rre_v.at[0, 0], rre_v.at[0, 1], rre_v.at[0, 2]),
            (rre_v.at[1, 0], rre_v.at[1, 1], rre_v.at[1, 2]))

    def issue(k, b):
        base = k * C
        bfs = bufs[b]
        pltpu.async_copy(xl_hbm.at[srcf.at[pl.ds(base, C)]], bfs[0], sems[b])
        pltpu.async_copy(xr_hbm.at[dstf.at[pl.ds(base, C)]], bfs[1], sems[b])
        pltpu.async_copy(eemb3_hbm.at[wid * NCHUNK + k], bfs[2], sems[b])

    def wait(b):
        pltpu.make_async_copy(eemb3_hbm.at[pl.ds(0, 3)], rre_v.at[b],
                              sems[b]).wait()

    lane = lax.iota(jnp.int32, 16)
    wsems = (sw0, sw1)

    def compute(k, b, wait_w):
        rl, rr, em = bufs[b]
        base = k * C
        if wait_w is not None:
            @pl.when(wait_w)
            def _():
                pltpu.make_async_copy(sdl_hbm.at[0], sdlrow_v.at[b],
                                      wsems[b]).wait()

        def block_body(v, _):
            lvec = jnp.zeros((16,), jnp.float32)
            for l in range(16):
                c = v * 16 + l
                acc = jnp.zeros((16,), jnp.float32)
                for j in range(D // 16):
                    sl = pl.ds(j * 16, 16)
                    m = rl[c, sl] + rr[c, sl] + em[c, sl]
                    a = jnp.where(m >= 0.0, m, 0.2 * m)
                    acc = acc + a * att_vecs[j]
                lvec = jnp.where(lane == l, jnp.sum(acc), lvec)
            sl16 = pl.ds(v * 16, 16)
            sdlrow_v[b, sl16] = srcf[pl.ds(base + v * 16, 16)]
            dvec = dstf[pl.ds(base + v * 16, 16)]
            sdlrow_v[b, pl.ds(C + v * 16, 16)] = dvec
            sdlrow_v[b, pl.ds(2 * C + v * 16, 16)] = plsc.bitcast(
                lvec, jnp.int32)
            s_d, s_l = plsc.sort_key_val(dvec, lvec)
            segm = _seg_scan(s_d, s_l, jnp.maximum)
            last = _run_last(s_d)
            cur = plsc.load_gather(smax_v, [s_d])
            plsc.store_scatter(smax_v, [s_d], jnp.maximum(cur, segm),
                               mask=last)
            return 0
        lax.fori_loop(0, C // 16, block_body, 0)
        pltpu.async_copy(sdlrow_v.at[b], sdl_hbm.at[wid * NCHUNK + k],
                         wsems[b])

    issue(0, 0)

    def pipe_body(i, _):
        k0 = 2 * i
        issue(k0 + 1, 1)
        wait(0)
        compute(k0, 0, k0 >= 2)
        issue(k0 + 2, 0)
        wait(1)
        compute(k0 + 1, 1, k0 >= 1)
        return 0
    lax.fori_loop(0, (NCHUNK - 1) // 2, pipe_body, 0)
    wait(0)
    compute(NCHUNK - 1, 0, jnp.bool_(True))
    pltpu.make_async_copy(sdl_hbm.at[0], sdlrow_v.at[0], sw0).wait()
    pltpu.make_async_copy(sdl_hbm.at[0], sdlrow_v.at[1], sw1).wait()

    pltpu.sync_copy(smax_v, shared_max.at[sid])
    plsc.subcore_barrier()
    per = NPAD // NS
    for j in range(NS):
        pltpu.sync_copy(shared_max.at[j, pl.ds(sid * per, per)], comb_v.at[j])

    def comb_body(v, _):
        sl = pl.ds(v * 16, 16)
        m = comb_v[0, sl]
        for j in range(1, NS):
            m = jnp.maximum(m, comb_v[j, sl])
        combo_v[sl] = m
        return 0
    lax.fori_loop(0, per // 16, comb_body, 0)
    pltpu.sync_copy(combo_v, smaxc_hbm.at[cid, pl.ds(sid * per, per)])


def _scb_body(xl_hbm, sdl_hbm, smaxc_hbm, smaxr_hbm,
              parts_hbm, denp_hbm,
              sdl_v, src_v, dstA, dstB, ex_v, rl_v, segA, den_v, acc_sh,
              si0, si1, s0, s1, sc0, sc1):
    cid = lax.axis_index("c")
    sid = lax.axis_index("s")
    wid = cid * NS + sid

    pltpu.sync_copy(smaxc_hbm.at[0], segA)
    pltpu.sync_copy(smaxr_hbm.at[1], rl_v.at[1])

    def comb_body(r, _):
        for j in range(D // 16):
            sl16 = pl.ds(r * D + j * 16, 16)
            segA[sl16] = jnp.maximum(segA[sl16],
                                     rl_v[1, r, pl.ds(j * 16, 16)])
        return 0
    lax.fori_loop(0, C, comb_body, 0)

    def zden_body(v, _):
        den_v[pl.ds(v * 16, 16)] = jnp.zeros((16,), jnp.float32)
        return 0
    lax.fori_loop(0, NPAD // 16, zden_body, 0)

    lane16 = lax.iota(jnp.int32, 16)

    def zrow_body(v, _):
        for j in range(D // 16):
            rl_v[0, v, pl.ds(j * 16, 16)] = jnp.zeros((16,), jnp.float32)
        return 0
    lax.fori_loop(0, C, zrow_body, 0)
    for t in range(NPAD // NS // C):
        for v in range(C // 16):
            dstA[pl.ds(v * 16, 16)] = (sid * (NPAD // NS) + t * C
                                       + v * 16 + lane16)
        pltpu.sync_copy(rl_v.at[0], acc_sh.at[dstA])
    plsc.subcore_barrier()

    isems = (si0, si1)
    rsems = (s0, s1)
    csems = (sc0, sc1)
    dst_bufs = (dstA, dstB)

    def issue_sdl(k, b):
        pltpu.async_copy(sdl_hbm.at[wid * NCHUNK + k], sdl_v.at[b], isems[b])

    def wait_sdl(b):
        pltpu.make_async_copy(sdl_hbm.at[0], sdl_v.at[b],
                              isems[b]).wait()

    def issue_rows(k, b, bi):
        for v in range(C // 16):
            src_v[pl.ds(v * 16, 16)] = sdl_v[bi, pl.ds(v * 16, 16)]
        pltpu.async_copy(xl_hbm.at[src_v], rl_v.at[b], rsems[b])

    def wait_rows(b):
        pltpu.make_async_copy(xl_hbm.at[pl.ds(0, C)], rl_v.at[b],
                              rsems[b]).wait()

    def step(k, p, do_sdl, do_next, wait_sc):
        dst_p = dst_bufs[p]
        if wait_sc is not None:
            @pl.when(wait_sc)
            def _():
                pltpu.make_async_copy(xl_hbm.at[pl.ds(0, C)],
                                      rl_v.at[p], csems[p]).wait()

        def ex_body(v, _):
            sl = pl.ds(v * 16, 16)
            dvec = sdl_v[p, pl.ds(C + v * 16, 16)]
            dst_p[sl] = dvec
            mx = plsc.load_gather(segA, [dvec])
            lg = plsc.bitcast(sdl_v[p, pl.ds(2 * C + v * 16, 16)],
                              jnp.float32)
            ex = jnp.exp(lg - mx)
            ex_v[sl] = ex
            s_d, s_e = plsc.sort_key_val(dvec, ex)
            sege = _seg_scan(s_d, s_e, lax.add)
            plsc.addupdate_scatter(den_v, [s_d], sege, mask=_run_last(s_d))
            return 0
        lax.fori_loop(0, C // 16, ex_body, 0)
        wait_rows(p)
        if do_sdl is not None:
            @pl.when(do_sdl)
            def _():
                issue_sdl(k + 2, p)
        if do_next:
            wait_sdl(1 - p)
            issue_rows(k + 1, 1 - p, 1 - p)

        def edge_block(v, _):
            exvec = ex_v[pl.ds(v * 16, 16)]
            for l in range(16):
                c = v * 16 + l
                e = exvec[l]
                for j in range(D // 16):
                    sl = pl.ds(j * 16, 16)
                    rl_v[p, c, sl] = rl_v[p, c, sl] * e
            return 0
        lax.fori_loop(0, C // 16, edge_block, 0)

        pltpu.async_copy(rl_v.at[p], acc_sh.at[dst_p], csems[p], add=True)

    issue_sdl(0, 0)
    wait_sdl(0)
    issue_rows(0, 0, 0)
    issue_sdl(1, 1)

    def pipe_body(i, _):
        k0 = 2 * i
        step(k0, 0, jnp.bool_(True), True, k0 >= 2)
        step(k0 + 1, 1, (k0 + 3) < NCHUNK, True, k0 >= 1)
        return 0
    lax.fori_loop(0, (NCHUNK - 1) // 2, pipe_body, 0)
    step(NCHUNK - 1, 0, None, False, jnp.bool_(True))

    pltpu.make_async_copy(xl_hbm.at[pl.ds(0, C)], rl_v.at[0], sc0).wait()
    pltpu.make_async_copy(xl_hbm.at[pl.ds(0, C)], rl_v.at[1], sc1).wait()

    pltpu.sync_copy(den_v, denp_hbm.at[wid])
    plsc.subcore_barrier()

    @pl.when(sid == 0)
    def _():
        pltpu.sync_copy(acc_sh, parts_hbm.at[cid])


@jax.jit
def kernel(inputs, hidden_states, edge_index, edge_attr, W_e, b_e, W_ih, b_ih,
           W_hh, b_hh, W_l, W_r, W_edge, att, b_gat, W_res, W_q, b_q):
    del hidden_states, W_ih, b_ih, W_hh, b_hh
    f32 = jnp.float32
    src = edge_index[0]
    dst = edge_index[1]

    RB = 2000
    xl, xr, xres = pl.pallas_call(
        _tc1_body,
        grid=(N // RB,),
        in_specs=[
            pl.BlockSpec((RB, D), lambda i: (i, 0)),
            pl.BlockSpec((D, D), lambda i: (0, 0)),
            pl.BlockSpec((8, D), lambda i: (0, 0)),
            pl.BlockSpec((D, D), lambda i: (0, 0)),
            pl.BlockSpec((D, D), lambda i: (0, 0)),
            pl.BlockSpec((D, D), lambda i: (0, 0)),
        ],
        out_specs=[
            pl.BlockSpec((RB, D), lambda i: (i, 0)),
            pl.BlockSpec((RB, D), lambda i: (i, 0)),
            pl.BlockSpec((RB, D), lambda i: (i, 0)),
        ],
        out_shape=[
            jax.ShapeDtypeStruct((N, D), f32),
            jax.ShapeDtypeStruct((N, D), f32),
            jax.ShapeDtypeStruct((N, D), f32),
        ],
    )(inputs, W_e, jnp.broadcast_to(b_e.reshape(1, D), (8, D)),
      W_l, W_r, W_res)

    EB = 16000
    e_emb = pl.pallas_call(
        _tc2_body,
        grid=(E // EB,),
        in_specs=[
            pl.BlockSpec((5, EB), lambda i: (0, i)),
            pl.BlockSpec((5, D), lambda i: (0, 0)),
        ],
        out_specs=pl.BlockSpec((EB, D), lambda i: (i, 0)),
        out_shape=jax.ShapeDtypeStruct((E, D), f32),
    )(jnp.transpose(edge_attr), W_edge)

    mesh = plsc.VectorSubcoreMesh(core_axis_name="c", subcore_axis_name="s")
    sc_params = pltpu.CompilerParams(needs_layout_passes=False)

    src2 = src.reshape(NW, EPW)
    dst2 = dst.reshape(NW, EPW)
    sdl, smaxc = pl.kernel(
        _sca_body,
        compiler_params=sc_params,
        out_type=(jax.ShapeDtypeStruct((NW * NCHUNK, 3 * C), jnp.int32),
                  jax.ShapeDtypeStruct((NC, NPAD), f32)),
        mesh=mesh,
        scratch_types=[
            pltpu.VMEM((EPW,), jnp.int32),
            pltpu.VMEM((EPW,), jnp.int32),
            pltpu.VMEM((2, 3, C, D), f32),
            pltpu.VMEM((2, 3 * C), jnp.int32),
            pltpu.VMEM((D,), f32),
            pltpu.VMEM((NPAD,), f32),
            pltpu.VMEM((NS, NPAD // NS), f32),
            pltpu.VMEM((NPAD // NS,), f32),
            pltpu.VMEM_SHARED((NS, NPAD), f32),
            pltpu.SemaphoreType.DMA,
            pltpu.SemaphoreType.DMA,
            pltpu.SemaphoreType.DMA,
            pltpu.SemaphoreType.DMA,
        ],
    )(xl, xr, e_emb.reshape(E // C, C, D), src2, dst2, att)

    parts, denp = pl.kernel(
        _scb_body,
        compiler_params=sc_params,
        out_type=(jax.ShapeDtypeStruct((NC, NPAD, D), f32),
                  jax.ShapeDtypeStruct((NW, NPAD), f32)),
        mesh=mesh,
        scratch_types=[
            pltpu.VMEM((2, 3 * C), jnp.int32),
            pltpu.VMEM((C,), jnp.int32),
            pltpu.VMEM((C,), jnp.int32),
            pltpu.VMEM((C,), jnp.int32),
            pltpu.VMEM((C,), f32),
            pltpu.VMEM((2, C, D), f32),
            pltpu.VMEM((NPAD,), f32),
            pltpu.VMEM((NPAD,), f32),
            pltpu.VMEM_SHARED((NPAD, D), f32),
            pltpu.SemaphoreType.DMA,
            pltpu.SemaphoreType.DMA,
            pltpu.SemaphoreType.DMA,
            pltpu.SemaphoreType.DMA,
            pltpu.SemaphoreType.DMA,
            pltpu.SemaphoreType.DMA,
        ],
    )(xl, sdl, smaxc, smaxc.reshape(NC, C, D))

    RB3 = 1280
    qp, hp = pl.pallas_call(
        _tc3_body,
        grid=(NPAD // RB3,),
        in_specs=[
            pl.BlockSpec((NC, RB3, D), lambda i: (0, i, 0)),
            pl.BlockSpec((NW, RB3), lambda i: (0, i)),
            pl.BlockSpec((RB3, D), lambda i: (i, 0)),
            pl.BlockSpec((8, D), lambda i: (0, 0)),
            pl.BlockSpec((D, A), lambda i: (0, 0)),
            pl.BlockSpec((8, A), lambda i: (0, 0)),
        ],
        out_specs=[
            pl.BlockSpec((RB3, A), lambda i: (i, 0)),
            pl.BlockSpec((RB3, D), lambda i: (i, 0)),
        ],
        out_shape=[
            jax.ShapeDtypeStruct((NPAD, A), f32),
            jax.ShapeDtypeStruct((NPAD, D), f32),
        ],
    )(parts, denp, jnp.pad(xres, ((0, NPAD - N), (0, 0))),
      jnp.broadcast_to(b_gat.reshape(1, D), (8, D)),
      W_q, jnp.broadcast_to(b_q.reshape(1, A), (8, A)))

    return (qp[:N], hp[:N])


# speedup vs baseline: 1.0213x; 1.0047x over previous
import jax
import jax.numpy as jnp
from jax import lax
from jax.experimental import pallas as pl
from jax.experimental.pallas import tpu as pltpu
from jax.experimental.pallas import tpu_sc as plsc

N = 10000
E = 320000
D = 128
A = 32
NPAD = 10240
WAUG = 144
NC = 2
NS = 16
NW = NC * NS
EPW = E // NW
C = 80
NCHUNK = EPW // C
SENT = -3.0e38


def _tc1_body(inp, we, be, wl, wr, wres, xl, xr, xres):
    x = jnp.maximum(jnp.dot(inp[...], we[...],
                            preferred_element_type=jnp.float32) + be[0:1], 0.0)
    xl[...] = jnp.dot(x, wl[...], preferred_element_type=jnp.float32)
    xr[...] = jnp.dot(x, wr[...], preferred_element_type=jnp.float32)
    xres[...] = jnp.dot(x, wres[...], preferred_element_type=jnp.float32)


def _tc2_body(eaT, wedge, out):
    out[...] = lax.dot_general(eaT[...], wedge[...],
                               (((0,), (0,)), ((), ())),
                               preferred_element_type=jnp.float32)


def _tc3_body(parts, denr, xresr, bg, wq, bq, qo, ho):
    num = parts[0] + parts[1]
    den = lax.dot_general(denr[...], jnp.ones((NW, 1), jnp.float32),
                          (((0,), (0,)), ((), ())),
                          preferred_element_type=jnp.float32)
    out = num / (den + 1e-16)
    h = out + xresr[...] + bg[0:1]
    ho[...] = h
    qo[...] = jnp.dot(h, wq[...], preferred_element_type=jnp.float32) + bq[0:1]


def _lane():
    return lax.iota(jnp.int32, 16)


def _take16(x, idx):
    dnums = lax.GatherDimensionNumbers(
        offset_dims=(), collapsed_slice_dims=(0,), start_index_map=(0,))
    return lax.gather(x, idx[:, None], dnums, slice_sizes=(1,),
                      mode=lax.GatherScatterMode.PROMISE_IN_BOUNDS)


def _shift_up(x, k):
    return _take16(x, jnp.maximum(_lane() - k, 0))


def _seg_scan(d, x, op):
    for k in (1, 2, 4, 8):
        xs = _shift_up(x, k)
        ds = _shift_up(d, k)
        x = jnp.where((ds == d) & (_lane() >= k), op(x, xs), x)
    return x


def _run_last(d):
    nxt = _take16(d, jnp.minimum(_lane() + 1, 15))
    return (d != nxt) | (_lane() == 15)


def _sca_body(xl_hbm, xr_hbm, eemb3_hbm, src_hbm, dst_hbm, att_hbm,
              sdl_hbm, smaxc_hbm,
              srcf, dstf, rre_v, sdlrow_v, att_v, smax_v,
              comb_v, combo_v, shared_max,
              sl0, sl1, sw0, sw1):
    cid = lax.axis_index("c")
    sid = lax.axis_index("s")
    wid = cid * NS + sid

    pltpu.sync_copy(att_hbm, att_v)
    att_vecs = [att_v[pl.ds(j * 16, 16)] for j in range(D // 16)]
    pltpu.sync_copy(src_hbm.at[wid], srcf)
    pltpu.sync_copy(dst_hbm.at[wid], dstf)

    def init_body(i, _):
        smax_v[pl.ds(i * 16, 16)] = jnp.full((16,), SENT, jnp.float32)
        return 0
    lax.fori_loop(0, NPAD // 16, init_body, 0)

    sems = (sl0, sl1)
    bufs = ((rre_v.at[0, 0], rre_v.at[0, 1], rre_v.at[0, 2]),
            (rre_v.at[1, 0], rre_v.at[1, 1], rre_v.at[1, 2]))

    def issue(k, b):
        base = k * C
        bfs = bufs[b]
        pltpu.async_copy(xl_hbm.at[srcf.at[pl.ds(base, C)]], bfs[0], sems[b])
        pltpu.async_copy(xr_hbm.at[dstf.at[pl.ds(base, C)]], bfs[1], sems[b])
        pltpu.async_copy(eemb3_hbm.at[wid * NCHUNK + k], bfs[2], sems[b])

    def wait(b):
        pltpu.make_async_copy(eemb3_hbm.at[pl.ds(0, 3)], rre_v.at[b],
                              sems[b]).wait()

    lane = lax.iota(jnp.int32, 16)
    wsems = (sw0, sw1)

    def compute(k, b, wait_w):
        rl, rr, em = bufs[b]
        base = k * C
        if wait_w is not None:
            @pl.when(wait_w)
            def _():
                pltpu.make_async_copy(sdl_hbm.at[0], sdlrow_v.at[b],
                                      wsems[b]).wait()

        def block_body(v, _):
            lvec = jnp.zeros((16,), jnp.float32)
            for l in range(16):
                c = v * 16 + l
                acc = jnp.zeros((16,), jnp.float32)
                for j in range(D // 16):
                    sl = pl.ds(j * 16, 16)
                    m = rl[c, sl] + rr[c, sl] + em[c, sl]
                    a = jnp.where(m >= 0.0, m, 0.2 * m)
                    acc = acc + a * att_vecs[j]
                lvec = jnp.where(lane == l, jnp.sum(acc), lvec)
            sl16 = pl.ds(v * 16, 16)
            sdlrow_v[b, sl16] = srcf[pl.ds(base + v * 16, 16)]
            dvec = dstf[pl.ds(base + v * 16, 16)]
            sdlrow_v[b, pl.ds(C + v * 16, 16)] = dvec
            sdlrow_v[b, pl.ds(2 * C + v * 16, 16)] = plsc.bitcast(
                lvec, jnp.int32)
            s_d, s_l = plsc.sort_key_val(dvec, lvec)
            segm = _seg_scan(s_d, s_l, jnp.maximum)
            last = _run_last(s_d)
            cur = plsc.load_gather(smax_v, [s_d])
            plsc.store_scatter(smax_v, [s_d], jnp.maximum(cur, segm),
                               mask=last)
            return 0
        lax.fori_loop(0, C // 16, block_body, 0)
        pltpu.async_copy(sdlrow_v.at[b], sdl_hbm.at[wid * NCHUNK + k],
                         wsems[b])

    issue(0, 0)

    def pipe_body(i, _):
        k0 = 2 * i
        issue(k0 + 1, 1)
        wait(0)
        compute(k0, 0, k0 >= 2)
        issue(k0 + 2, 0)
        wait(1)
        compute(k0 + 1, 1, k0 >= 1)
        return 0
    lax.fori_loop(0, (NCHUNK - 1) // 2, pipe_body, 0)
    wait(0)
    compute(NCHUNK - 1, 0, jnp.bool_(True))
    pltpu.make_async_copy(sdl_hbm.at[0], sdlrow_v.at[0], sw0).wait()
    pltpu.make_async_copy(sdl_hbm.at[0], sdlrow_v.at[1], sw1).wait()

    pltpu.sync_copy(smax_v, shared_max.at[sid])
    plsc.subcore_barrier()
    per = NPAD // NS
    for j in range(NS):
        pltpu.sync_copy(shared_max.at[j, pl.ds(sid * per, per)], comb_v.at[j])

    def comb_body(v, _):
        sl = pl.ds(v * 16, 16)
        m = comb_v[0, sl]
        for j in range(1, NS):
            m = jnp.maximum(m, comb_v[j, sl])
        combo_v[sl] = m
        return 0
    lax.fori_loop(0, per // 16, comb_body, 0)
    pltpu.sync_copy(combo_v, smaxc_hbm.at[cid, pl.ds(sid * per, per)])


def _scb_body(xl_hbm, sdl_hbm, smaxc_hbm, smaxr_hbm,
              parts_hbm, denp_hbm,
              sdl_v, src_v, dstA, dstB, ex_v, rl_v, segA, den_v, acc_sh,
              si0, si1, s0, s1, sc0, sc1):
    cid = lax.axis_index("c")
    sid = lax.axis_index("s")
    wid = cid * NS + sid

    pltpu.sync_copy(smaxc_hbm.at[0], segA)
    pltpu.sync_copy(smaxr_hbm.at[1], rl_v.at[1])

    def comb_body(r, _):
        for j in range(D // 16):
            sl16 = pl.ds(r * D + j * 16, 16)
            segA[sl16] = jnp.maximum(segA[sl16],
                                     rl_v[1, r, pl.ds(j * 16, 16)])
        return 0
    lax.fori_loop(0, C, comb_body, 0)

    def zden_body(v, _):
        den_v[pl.ds(v * 16, 16)] = jnp.zeros((16,), jnp.float32)
        return 0
    lax.fori_loop(0, NPAD // 16, zden_body, 0)

    lane16 = lax.iota(jnp.int32, 16)

    def zrow_body(v, _):
        for j in range(D // 16):
            rl_v[0, v, pl.ds(j * 16, 16)] = jnp.zeros((16,), jnp.float32)
        return 0
    lax.fori_loop(0, C, zrow_body, 0)
    for t in range(NPAD // NS // C):
        for v in range(C // 16):
            dstA[pl.ds(v * 16, 16)] = (sid * (NPAD // NS) + t * C
                                       + v * 16 + lane16)
        pltpu.sync_copy(rl_v.at[0], acc_sh.at[dstA])
    plsc.subcore_barrier()

    isems = (si0, si1)
    rsems = (s0, s1)
    csems = (sc0, sc1)
    dst_bufs = (dstA, dstB)

    def issue_sdl(k, b):
        pltpu.async_copy(sdl_hbm.at[wid * NCHUNK + k], sdl_v.at[b], isems[b])

    def wait_sdl(b):
        pltpu.make_async_copy(sdl_hbm.at[0], sdl_v.at[b],
                              isems[b]).wait()

    def issue_rows(k, b, bi):
        for v in range(C // 16):
            src_v[pl.ds(v * 16, 16)] = sdl_v[bi, pl.ds(v * 16, 16)]
        pltpu.async_copy(xl_hbm.at[src_v], rl_v.at[b], rsems[b])

    def wait_rows(b):
        pltpu.make_async_copy(xl_hbm.at[pl.ds(0, C)], rl_v.at[b],
                              rsems[b]).wait()

    def step(k, p, do_sdl, do_next, wait_sc):
        dst_p = dst_bufs[p]
        if wait_sc is not None:
            @pl.when(wait_sc)
            def _():
                pltpu.make_async_copy(xl_hbm.at[pl.ds(0, C)],
                                      rl_v.at[p], csems[p]).wait()

        def ex_body(v, _):
            sl = pl.ds(v * 16, 16)
            dvec = sdl_v[p, pl.ds(C + v * 16, 16)]
            dst_p[sl] = dvec
            mx = plsc.load_gather(segA, [dvec])
            lg = plsc.bitcast(sdl_v[p, pl.ds(2 * C + v * 16, 16)],
                              jnp.float32)
            ex = jnp.exp(lg - mx)
            ex_v[sl] = ex
            s_d, s_e = plsc.sort_key_val(dvec, ex)
            sege = _seg_scan(s_d, s_e, lax.add)
            plsc.addupdate_scatter(den_v, [s_d], sege, mask=_run_last(s_d))
            return 0
        lax.fori_loop(0, C // 16, ex_body, 0)
        wait_rows(p)
        if do_sdl is not None:
            @pl.when(do_sdl)
            def _():
                issue_sdl(k + 2, p)
        if do_next:
            wait_sdl(1 - p)
            issue_rows(k + 1, 1 - p, 1 - p)

        def edge_block(v, _):
            exvec = ex_v[pl.ds(v * 16, 16)]
            for l in range(16):
                c = v * 16 + l
                e = exvec[l]
                for j in range(D // 16):
                    sl = pl.ds(j * 16, 16)
                    rl_v[p, c, sl] = rl_v[p, c, sl] * e
            return 0
        lax.fori_loop(0, C // 16, edge_block, 0)

        pltpu.async_copy(rl_v.at[p], acc_sh.at[dst_p], csems[p], add=True)

    issue_sdl(0, 0)
    wait_sdl(0)
    issue_rows(0, 0, 0)
    issue_sdl(1, 1)

    def pipe_body(i, _):
        k0 = 2 * i
        step(k0, 0, jnp.bool_(True), True, k0 >= 2)
        step(k0 + 1, 1, (k0 + 3) < NCHUNK, True, k0 >= 1)
        return 0
    lax.fori_loop(0, (NCHUNK - 1) // 2, pipe_body, 0)
    step(NCHUNK - 1, 0, None, False, jnp.bool_(True))

    pltpu.make_async_copy(xl_hbm.at[pl.ds(0, C)], rl_v.at[0], sc0).wait()
    pltpu.make_async_copy(xl_hbm.at[pl.ds(0, C)], rl_v.at[1], sc1).wait()

    pltpu.sync_copy(den_v, denp_hbm.at[wid])
    plsc.subcore_barrier()

    @pl.when(sid == 0)
    def _():
        pltpu.sync_copy(acc_sh, parts_hbm.at[cid])


@jax.jit
def kernel(inputs, hidden_states, edge_index, edge_attr, W_e, b_e, W_ih, b_ih,
           W_hh, b_hh, W_l, W_r, W_edge, att, b_gat, W_res, W_q, b_q):
    del hidden_states, W_ih, b_ih, W_hh, b_hh
    f32 = jnp.float32
    src = edge_index[0]
    dst = edge_index[1]

    RB = 2000
    xl, xr, xres = pl.pallas_call(
        _tc1_body,
        grid=(N // RB,),
        in_specs=[
            pl.BlockSpec((RB, D), lambda i: (i, 0)),
            pl.BlockSpec((D, D), lambda i: (0, 0)),
            pl.BlockSpec((8, D), lambda i: (0, 0)),
            pl.BlockSpec((D, D), lambda i: (0, 0)),
            pl.BlockSpec((D, D), lambda i: (0, 0)),
            pl.BlockSpec((D, D), lambda i: (0, 0)),
        ],
        out_specs=[
            pl.BlockSpec((RB, D), lambda i: (i, 0)),
            pl.BlockSpec((RB, D), lambda i: (i, 0)),
            pl.BlockSpec((RB, D), lambda i: (i, 0)),
        ],
        out_shape=[
            jax.ShapeDtypeStruct((N, D), f32),
            jax.ShapeDtypeStruct((N, D), f32),
            jax.ShapeDtypeStruct((N, D), f32),
        ],
    )(inputs, W_e, jnp.broadcast_to(b_e.reshape(1, D), (8, D)),
      W_l, W_r, W_res)

    EB = 32000
    e_emb = pl.pallas_call(
        _tc2_body,
        grid=(E // EB,),
        in_specs=[
            pl.BlockSpec((5, EB), lambda i: (0, i)),
            pl.BlockSpec((5, D), lambda i: (0, 0)),
        ],
        out_specs=pl.BlockSpec((EB, D), lambda i: (i, 0)),
        out_shape=jax.ShapeDtypeStruct((E, D), f32),
    )(jnp.transpose(edge_attr), W_edge)

    mesh = plsc.VectorSubcoreMesh(core_axis_name="c", subcore_axis_name="s")
    sc_params = pltpu.CompilerParams(needs_layout_passes=False)

    src2 = src.reshape(NW, EPW)
    dst2 = dst.reshape(NW, EPW)
    sdl, smaxc = pl.kernel(
        _sca_body,
        compiler_params=sc_params,
        out_type=(jax.ShapeDtypeStruct((NW * NCHUNK, 3 * C), jnp.int32),
                  jax.ShapeDtypeStruct((NC, NPAD), f32)),
        mesh=mesh,
        scratch_types=[
            pltpu.VMEM((EPW,), jnp.int32),
            pltpu.VMEM((EPW,), jnp.int32),
            pltpu.VMEM((2, 3, C, D), f32),
            pltpu.VMEM((2, 3 * C), jnp.int32),
            pltpu.VMEM((D,), f32),
            pltpu.VMEM((NPAD,), f32),
            pltpu.VMEM((NS, NPAD // NS), f32),
            pltpu.VMEM((NPAD // NS,), f32),
            pltpu.VMEM_SHARED((NS, NPAD), f32),
            pltpu.SemaphoreType.DMA,
            pltpu.SemaphoreType.DMA,
            pltpu.SemaphoreType.DMA,
            pltpu.SemaphoreType.DMA,
        ],
    )(xl, xr, e_emb.reshape(E // C, C, D), src2, dst2, att)

    parts, denp = pl.kernel(
        _scb_body,
        compiler_params=sc_params,
        out_type=(jax.ShapeDtypeStruct((NC, NPAD, D), f32),
                  jax.ShapeDtypeStruct((NW, NPAD), f32)),
        mesh=mesh,
        scratch_types=[
            pltpu.VMEM((2, 3 * C), jnp.int32),
            pltpu.VMEM((C,), jnp.int32),
            pltpu.VMEM((C,), jnp.int32),
            pltpu.VMEM((C,), jnp.int32),
            pltpu.VMEM((C,), f32),
            pltpu.VMEM((2, C, D), f32),
            pltpu.VMEM((NPAD,), f32),
            pltpu.VMEM((NPAD,), f32),
            pltpu.VMEM_SHARED((NPAD, D), f32),
            pltpu.SemaphoreType.DMA,
            pltpu.SemaphoreType.DMA,
            pltpu.SemaphoreType.DMA,
            pltpu.SemaphoreType.DMA,
            pltpu.SemaphoreType.DMA,
            pltpu.SemaphoreType.DMA,
        ],
    )(xl, sdl, smaxc, smaxc.reshape(NC, C, D))

    RB3 = 1280
    qp, hp = pl.pallas_call(
        _tc3_body,
        grid=(NPAD // RB3,),
        in_specs=[
            pl.BlockSpec((NC, RB3, D), lambda i: (0, i, 0)),
            pl.BlockSpec((NW, RB3), lambda i: (0, i)),
            pl.BlockSpec((RB3, D), lambda i: (i, 0)),
            pl.BlockSpec((8, D), lambda i: (0, 0)),
            pl.BlockSpec((D, A), lambda i: (0, 0)),
            pl.BlockSpec((8, A), lambda i: (0, 0)),
        ],
        out_specs=[
            pl.BlockSpec((RB3, A), lambda i: (i, 0)),
            pl.BlockSpec((RB3, D), lambda i: (i, 0)),
        ],
        out_shape=[
            jax.ShapeDtypeStruct((NPAD, A), f32),
            jax.ShapeDtypeStruct((NPAD, D), f32),
        ],
    )(parts, denp, jnp.pad(xres, ((0, NPAD - N), (0, 0))),
      jnp.broadcast_to(b_gat.reshape(1, D), (8, D)),
      W_q, jnp.broadcast_to(b_q.reshape(1, A), (8, A)))

    return (qp[:N], hp[:N])
